# Initial kernel scaffold; baseline (speedup 1.0000x reference)
import jax
import jax.numpy as jnp
from jax.experimental import pallas as pl


def kernel(feat, nodes, neigh_idx, W_self, W_rel):
    raise NotImplementedError("write your pallas kernel here")


# speedup vs baseline: 1.8300x; 1.8300x over previous
import functools

import jax
import jax.numpy as jnp
from jax import lax
from jax.experimental import pallas as pl
from jax.experimental.pallas import tpu as pltpu
from jax.experimental.pallas import tpu_sc as plsc

N_NODES = 100000
D = 128
D_OUT = 128
B = 20000
K = 25

NC = 2
NS = 16
NW = NC * NS
BP = 20480
PER_W = BP // NW
C = 128
NCH = PER_W // C


def _sc_body(feat, nodesp, neigh_t, xs_out, xsum_out,
             idxv, sidx, planes, selfbuf, acc, sem_s, sem_p0, sem_p1):
    wid = lax.axis_index("s") * NC + lax.axis_index("c")

    def chunk(i, carry):
        base = wid * PER_W + i * C
        pltpu.sync_copy(neigh_t.at[:, pl.ds(base, C)], idxv)
        pltpu.sync_copy(nodesp.at[pl.ds(base, C)], sidx)
        cp_self = pltpu.async_copy(feat.at[sidx], selfbuf, sem_s)
        sems = [sem_p0, sem_p1]
        cps = [None, None]
        cps[0] = pltpu.async_copy(feat.at[idxv.at[0]], planes.at[0], sems[0])
        for k in range(K):
            if k + 1 < K:
                nb = (k + 1) % 2
                cps[nb] = pltpu.async_copy(feat.at[idxv.at[k + 1]],
                                           planes.at[nb], sems[nb])
            cps[k % 2].wait()
            pb = k % 2
            first = k == 0

            def body(c, _):
                for j in range(8):
                    v = planes[pb, c, pl.ds(j * 16, 16)]
                    if first:
                        acc[c, pl.ds(j * 16, 16)] = v
                    else:
                        plsc.addupdate(acc.at[c, pl.ds(j * 16, 16)], v)
                return 0

            lax.fori_loop(0, C, body, 0, unroll=4)
        cp_self.wait()
        pltpu.sync_copy(acc, xsum_out.at[pl.ds(base, C)])
        pltpu.sync_copy(selfbuf, xs_out.at[pl.ds(base, C)])
        return carry

    lax.fori_loop(0, NCH, chunk, 0)


def _sc_gather(feat, nodesp, neigh_t):
    mesh = plsc.VectorSubcoreMesh(core_axis_name="c", subcore_axis_name="s")
    f = pl.kernel(
        _sc_body, mesh=mesh,
        out_type=(jax.ShapeDtypeStruct((BP, D), jnp.float32),
                  jax.ShapeDtypeStruct((BP, D), jnp.float32)),
        scratch_types=[
            pltpu.VMEM((K, C), jnp.int32),
            pltpu.VMEM((C,), jnp.int32),
            pltpu.VMEM((2, C, D), jnp.float32),
            pltpu.VMEM((C, D), jnp.float32),
            pltpu.VMEM((C, D), jnp.float32),
            pltpu.SemaphoreType.DMA,
            pltpu.SemaphoreType.DMA,
            pltpu.SemaphoreType.DMA,
        ],
    )
    return f(feat, nodesp, neigh_t)


def _mm_body(xs_ref, xm_ref, ws_ref, wr_ref, o_ref):
    a = lax.dot_general(ws_ref[...], xs_ref[...],
                        (((1,), (1,)), ((), ())),
                        preferred_element_type=jnp.float32)
    b = lax.dot_general(wr_ref[...], xm_ref[...],
                        (((1,), (1,)), ((), ())),
                        preferred_element_type=jnp.float32)
    o_ref[...] = jnp.maximum(0.5 * a + (0.5 / K) * b, 0.0)


def _tc_combine(xs, xm, w_self, w_rel):
    tb = 2560
    grid = BP // tb
    return pl.pallas_call(
        _mm_body,
        grid=(grid,),
        in_specs=[
            pl.BlockSpec((tb, D), lambda i: (i, 0)),
            pl.BlockSpec((tb, D), lambda i: (i, 0)),
            pl.BlockSpec((D_OUT, D), lambda i: (0, 0)),
            pl.BlockSpec((D_OUT, D), lambda i: (0, 0)),
        ],
        out_specs=pl.BlockSpec((D_OUT, tb), lambda i: (0, i)),
        out_shape=jax.ShapeDtypeStruct((D_OUT, BP), jnp.float32),
    )(xs, xm, w_self, w_rel)


@jax.jit
def kernel(feat, nodes, neigh_idx, W_self, W_rel):
    nodesp = jnp.pad(nodes, (0, BP - B))
    neigh_t = jnp.pad(neigh_idx, ((0, BP - B), (0, 0))).T.copy()
    xs, xsum = _sc_gather(feat, nodesp, neigh_t)
    out = _tc_combine(xs, xsum, W_self, W_rel)
    return out[:, :B]
